# baseline (device time: 113448 ns/iter reference)
import jax
import jax.numpy as jnp
from jax import lax
from jax.experimental import pallas as pl
from jax.experimental.pallas import tpu as pltpu

N_DEV = 4
SQ = 1024
SKV = 1024
HQ_SHARD = 8
DH = 128
BLK = 64
SCALE = 0.08838834764831843
CHUNK = SQ // N_DEV
N_HOPS = 2 * (N_DEV - 1)


def kernel(x, Wq, K_ext, V_ext, Wo):
    my_i = lax.axis_index("i")
    x2 = x[0]
    K = lax.dynamic_slice_in_dim(K_ext[0], my_i * HQ_SHARD, HQ_SHARD, axis=1)
    V = lax.dynamic_slice_in_dim(V_ext[0], my_i * HQ_SHARD, HQ_SHARD, axis=1)
    K = jnp.transpose(K, (1, 0, 2))
    V = jnp.transpose(V, (1, 0, 2))

    def body(x_ref, wq_ref, k_ref, v_ref, wo_ref, out_ref,
             comm_ref, send_sems, recv_sems, ctx_ref):
        my = lax.axis_index("i")
        left = lax.rem(my + N_DEV - 1, N_DEV)
        right = lax.rem(my + 1, N_DEV)

        barrier_sem = pltpu.get_barrier_semaphore()
        for nbr in (left, right):
            pl.semaphore_signal(barrier_sem, inc=1, device_id=(nbr,),
                                device_id_type=pl.DeviceIdType.MESH)
        pl.semaphore_wait(barrier_sem, 2)

        q = jnp.dot(x_ref[...], wq_ref[...],
                    preferred_element_type=jnp.float32)

        rb = lax.broadcasted_iota(jnp.int32, (SQ, SKV), 0) // BLK
        cb = lax.broadcasted_iota(jnp.int32, (SQ, SKV), 1) // BLK
        mask = cb <= rb

        for h in range(HQ_SHARD):
            qh = q[:, h * DH:(h + 1) * DH]
            s = lax.dot_general(qh, k_ref[h],
                                (((1,), (1,)), ((), ())),
                                preferred_element_type=jnp.float32) * SCALE
            s = jnp.where(mask, s, -1e9)
            m = jnp.max(s, axis=-1, keepdims=True)
            w = jnp.exp(s - m)
            w = w / jnp.sum(w, axis=-1, keepdims=True)
            ctx_ref[:, h * DH:(h + 1) * DH] = jnp.dot(
                w, v_ref[h], preferred_element_type=jnp.float32)

        partial = jnp.dot(ctx_ref[...], wo_ref[...],
                          preferred_element_type=jnp.float32)
        for c in range(N_DEV):
            out_ref[c] = partial[c * CHUNK:(c + 1) * CHUNK, :]


        for hop in range(N_DEV - 1):
            send_idx = lax.rem(my - hop + N_DEV, N_DEV)
            recv_idx = lax.rem(my - hop - 1 + N_DEV, N_DEV)
            rdma = pltpu.make_async_remote_copy(
                src_ref=out_ref.at[send_idx],
                dst_ref=comm_ref.at[hop],
                send_sem=send_sems.at[hop],
                recv_sem=recv_sems.at[hop],
                device_id=(right,),
                device_id_type=pl.DeviceIdType.MESH,
            )
            rdma.start()
            rdma.wait()
            out_ref[recv_idx] = out_ref[recv_idx] + comm_ref[hop]

        for g in range(N_DEV - 1):
            hop = N_DEV - 1 + g
            send_idx = lax.rem(my + 1 - g + N_DEV, N_DEV)
            recv_idx = lax.rem(my - g + N_DEV, N_DEV)
            rdma = pltpu.make_async_remote_copy(
                src_ref=out_ref.at[send_idx],
                dst_ref=comm_ref.at[hop],
                send_sem=send_sems.at[hop],
                recv_sem=recv_sems.at[hop],
                device_id=(right,),
                device_id_type=pl.DeviceIdType.MESH,
            )
            rdma.start()
            rdma.wait()
            out_ref[recv_idx] = comm_ref[hop]

    out = pl.pallas_call(
        body,
        out_shape=jax.ShapeDtypeStruct((N_DEV, CHUNK, SQ), jnp.float32),
        in_specs=[pl.BlockSpec(memory_space=pltpu.VMEM)] * 5,
        out_specs=pl.BlockSpec(memory_space=pltpu.VMEM),
        scratch_shapes=[
            pltpu.VMEM((N_HOPS, CHUNK, SQ), jnp.float32),
            pltpu.SemaphoreType.DMA((N_HOPS,)),
            pltpu.SemaphoreType.DMA((N_HOPS,)),
            pltpu.VMEM((SQ, HQ_SHARD * DH), jnp.float32),
        ],
        compiler_params=pltpu.CompilerParams(collective_id=0),
    )(x2, Wq, K, V, Wo)
    return out.reshape(1, SQ, SQ)


# device time: 80052 ns/iter; 1.4172x vs baseline; 1.4172x over previous
import jax
import jax.numpy as jnp
from jax import lax
from jax.experimental import pallas as pl
from jax.experimental.pallas import tpu as pltpu

N_DEV = 4
SQ = 1024
SKV = 1024
HQ_SHARD = 8
DH = 128
BLK = 64
SCALE = 0.08838834764831843
N_CHUNKS = 2 * N_DEV
CHUNK = SQ // N_CHUNKS
N_HOPS = 2 * (N_DEV - 1)


def kernel(x, Wq, K_ext, V_ext, Wo):
    my_i = lax.axis_index("i")
    x2 = x[0]
    K = lax.dynamic_slice_in_dim(K_ext[0], my_i * HQ_SHARD, HQ_SHARD, axis=1)
    V = lax.dynamic_slice_in_dim(V_ext[0], my_i * HQ_SHARD, HQ_SHARD, axis=1)
    K = jnp.transpose(K, (1, 0, 2))
    V = jnp.transpose(V, (1, 0, 2))

    def body(x_ref, wq_ref, k_ref, v_ref, wo_ref, out_ref,
             comm_ref, send_sems, recv_sems, ctx_ref):
        my = lax.axis_index("i")
        left = lax.rem(my + N_DEV - 1, N_DEV)
        right = lax.rem(my + 1, N_DEV)

        barrier_sem = pltpu.get_barrier_semaphore()
        for nbr in (left, right):
            pl.semaphore_signal(barrier_sem, inc=1, device_id=(nbr,),
                                device_id_type=pl.DeviceIdType.MESH)
        pl.semaphore_wait(barrier_sem, 2)

        q = jnp.dot(x_ref[...], wq_ref[...],
                    preferred_element_type=jnp.float32)

        rb = lax.broadcasted_iota(jnp.int32, (SQ, SKV), 0) // BLK
        cb = lax.broadcasted_iota(jnp.int32, (SQ, SKV), 1) // BLK
        mask = cb <= rb

        for h in range(HQ_SHARD):
            qh = q[:, h * DH:(h + 1) * DH]
            s = lax.dot_general(qh, k_ref[h],
                                (((1,), (1,)), ((), ())),
                                preferred_element_type=jnp.float32) * SCALE
            s = jnp.where(mask, s, -1e9)
            m = jnp.max(s, axis=-1, keepdims=True)
            w = jnp.exp(s - m)
            w = w / jnp.sum(w, axis=-1, keepdims=True)
            ctx_ref[:, h * DH:(h + 1) * DH] = jnp.dot(
                w, v_ref[h], preferred_element_type=jnp.float32)

        partial = jnp.dot(ctx_ref[...], wo_ref[...],
                          preferred_element_type=jnp.float32)
        for c in range(N_CHUNKS):
            out_ref[c] = partial[c * CHUNK:(c + 1) * CHUNK, :]

        def make(send_idx, slot, dst_dev):
            return pltpu.make_async_remote_copy(
                src_ref=out_ref.at[send_idx],
                dst_ref=comm_ref.at[slot],
                send_sem=send_sems.at[slot],
                recv_sem=recv_sems.at[slot],
                device_id=(dst_dev,),
                device_id_type=pl.DeviceIdType.MESH,
            )

        for hop in range(N_DEV - 1):
            cw_send = lax.rem(my - hop + N_DEV, N_DEV)
            cw_recv = lax.rem(my - hop - 1 + N_DEV, N_DEV)
            ccw_send = N_DEV + lax.rem(my + hop, N_DEV)
            ccw_recv = N_DEV + lax.rem(my + hop + 1, N_DEV)
            cw = make(cw_send, hop, right)
            ccw = make(ccw_send, N_HOPS + hop, left)
            cw.start()
            ccw.start()
            cw.wait()
            ccw.wait()
            out_ref[cw_recv] = out_ref[cw_recv] + comm_ref[hop]
            out_ref[ccw_recv] = out_ref[ccw_recv] + comm_ref[N_HOPS + hop]

        for g in range(N_DEV - 1):
            hop = N_DEV - 1 + g
            cw_send = lax.rem(my + 1 - g + N_DEV, N_DEV)
            cw_recv = lax.rem(my - g + N_DEV, N_DEV)
            ccw_send = N_DEV + lax.rem(my - 1 + g + N_DEV, N_DEV)
            ccw_recv = N_DEV + lax.rem(my + g, N_DEV)
            cw = make(cw_send, hop, right)
            ccw = make(ccw_send, N_HOPS + hop, left)
            cw.start()
            ccw.start()
            cw.wait()
            ccw.wait()
            out_ref[cw_recv] = comm_ref[hop]
            out_ref[ccw_recv] = comm_ref[N_HOPS + hop]

    out = pl.pallas_call(
        body,
        out_shape=jax.ShapeDtypeStruct((N_CHUNKS, CHUNK, SQ), jnp.float32),
        in_specs=[pl.BlockSpec(memory_space=pltpu.VMEM)] * 5,
        out_specs=pl.BlockSpec(memory_space=pltpu.VMEM),
        scratch_shapes=[
            pltpu.VMEM((2 * N_HOPS, CHUNK, SQ), jnp.float32),
            pltpu.SemaphoreType.DMA((2 * N_HOPS,)),
            pltpu.SemaphoreType.DMA((2 * N_HOPS,)),
            pltpu.VMEM((SQ, HQ_SHARD * DH), jnp.float32),
        ],
        compiler_params=pltpu.CompilerParams(collective_id=0),
    )(x2, Wq, K, V, Wo)
    return out.reshape(1, SQ, SQ)


# device time: 79403 ns/iter; 1.4288x vs baseline; 1.0082x over previous
import jax
import jax.numpy as jnp
from jax import lax
from jax.experimental import pallas as pl
from jax.experimental.pallas import tpu as pltpu

N_DEV = 4
SQ = 1024
SKV = 1024
HQ_SHARD = 8
DH = 128
BLK = 64
SCALE = 0.08838834764831843
N_CHUNKS = 2 * N_DEV
CHUNK = SQ // N_CHUNKS
N_HOPS = 2 * (N_DEV - 1)


def kernel(x, Wq, K_ext, V_ext, Wo):
    my_i = lax.axis_index("i")
    x2 = x[0]
    K = lax.dynamic_slice_in_dim(K_ext[0], my_i * HQ_SHARD, HQ_SHARD, axis=1)
    V = lax.dynamic_slice_in_dim(V_ext[0], my_i * HQ_SHARD, HQ_SHARD, axis=1)
    K = jnp.transpose(K, (1, 0, 2))
    V = jnp.transpose(V, (1, 0, 2))

    def body(x_ref, wq_ref, k_ref, v_ref, wo_ref, out_ref,
             comm_ref, send_sems, recv_sems, ctx_ref):
        my = lax.axis_index("i")
        left = lax.rem(my + N_DEV - 1, N_DEV)
        right = lax.rem(my + 1, N_DEV)

        barrier_sem = pltpu.get_barrier_semaphore()
        for nbr in (left, right):
            pl.semaphore_signal(barrier_sem, inc=1, device_id=(nbr,),
                                device_id_type=pl.DeviceIdType.MESH)
        pl.semaphore_wait(barrier_sem, 2)

        cb = lax.broadcasted_iota(jnp.int32, (CHUNK, SKV), 1) // BLK
        rbase = lax.broadcasted_iota(jnp.int32, (CHUNK, SKV), 0)

        def compute_chunk(idx):
            r0 = idx * CHUNK
            xc = x_ref[pl.ds(r0, CHUNK), :]
            qc = jnp.dot(xc, wq_ref[...],
                         preferred_element_type=jnp.float32)
            mask = cb <= ((rbase + r0) // BLK)
            for h in range(HQ_SHARD):
                qh = qc[:, h * DH:(h + 1) * DH]
                s = lax.dot_general(qh, k_ref[h],
                                    (((1,), (1,)), ((), ())),
                                    preferred_element_type=jnp.float32) * SCALE
                s = jnp.where(mask, s, -1e9)
                m = jnp.max(s, axis=-1, keepdims=True)
                w = jnp.exp(s - m)
                w = w / jnp.sum(w, axis=-1, keepdims=True)
                ctx_ref[:, h * DH:(h + 1) * DH] = jnp.dot(
                    w, v_ref[h], preferred_element_type=jnp.float32)
            out_ref[idx] = jnp.dot(ctx_ref[...], wo_ref[...],
                                   preferred_element_type=jnp.float32)

        def cw_chunk(j):
            return lax.rem(my - j + N_DEV, N_DEV)

        def ccw_chunk(j):
            return N_DEV + lax.rem(my + j, N_DEV)

        def make(src, slot, dst_dev):
            return pltpu.make_async_remote_copy(
                src_ref=src,
                dst_ref=comm_ref.at[slot],
                send_sem=send_sems.at[slot],
                recv_sem=recv_sems.at[slot],
                device_id=(dst_dev,),
                device_id_type=pl.DeviceIdType.MESH,
            )

        compute_chunk(cw_chunk(0))
        compute_chunk(ccw_chunk(0))

        for hop in range(N_DEV - 1):
            cw = make(out_ref.at[cw_chunk(hop)], hop, right)
            ccw = make(out_ref.at[ccw_chunk(hop)], N_HOPS + hop, left)
            cw.start()
            ccw.start()
            compute_chunk(cw_chunk(hop + 1))
            compute_chunk(ccw_chunk(hop + 1))
            cw.wait_recv()
            ccw.wait_recv()
            cw_recv = cw_chunk(hop + 1)
            ccw_recv = ccw_chunk(hop + 1)
            out_ref[cw_recv] = out_ref[cw_recv] + comm_ref[hop]
            out_ref[ccw_recv] = out_ref[ccw_recv] + comm_ref[N_HOPS + hop]

        for g in range(N_DEV - 1):
            hop = N_DEV - 1 + g
            if g == 0:
                cw_src = out_ref.at[cw_chunk(N_DEV - 1)]
                ccw_src = out_ref.at[ccw_chunk(N_DEV - 1)]
            else:
                cw_src = comm_ref.at[hop - 1]
                ccw_src = comm_ref.at[N_HOPS + hop - 1]
            cw = make(cw_src, hop, right)
            ccw = make(ccw_src, N_HOPS + hop, left)
            cw.start()
            ccw.start()
            cw.wait_recv()
            ccw.wait_recv()
            out_ref[lax.rem(my - g + N_DEV, N_DEV)] = comm_ref[hop]
            out_ref[N_DEV + lax.rem(my + g, N_DEV)] = comm_ref[N_HOPS + hop]

        for slot in range(2 * N_HOPS):
            drain = make(comm_ref.at[slot], slot, right)
            drain.wait_send()

    out = pl.pallas_call(
        body,
        out_shape=jax.ShapeDtypeStruct((N_CHUNKS, CHUNK, SQ), jnp.float32),
        in_specs=[pl.BlockSpec(memory_space=pltpu.VMEM)] * 5,
        out_specs=pl.BlockSpec(memory_space=pltpu.VMEM),
        scratch_shapes=[
            pltpu.VMEM((2 * N_HOPS, CHUNK, SQ), jnp.float32),
            pltpu.SemaphoreType.DMA((2 * N_HOPS,)),
            pltpu.SemaphoreType.DMA((2 * N_HOPS,)),
            pltpu.VMEM((CHUNK, HQ_SHARD * DH), jnp.float32),
        ],
        compiler_params=pltpu.CompilerParams(collective_id=0),
    )(x2, Wq, K, V, Wo)
    return out.reshape(1, SQ, SQ)


# device time: 68209 ns/iter; 1.6632x vs baseline; 1.1641x over previous
import jax
import jax.numpy as jnp
from jax import lax
from jax.experimental import pallas as pl
from jax.experimental.pallas import tpu as pltpu

N_DEV = 4
SQ = 1024
SKV = 1024
HQ_SHARD = 8
DH = 128
BLK = 64
SCALE = 0.08838834764831843
N_CHUNKS = 2 * N_DEV
CHUNK = SQ // N_CHUNKS
N_HOPS = 2 * (N_DEV - 1)


def kernel(x, Wq, K_ext, V_ext, Wo):
    my_i = lax.axis_index("i")
    x2 = x[0]
    K = lax.dynamic_slice_in_dim(K_ext[0], my_i * HQ_SHARD, HQ_SHARD, axis=1)
    V = lax.dynamic_slice_in_dim(V_ext[0], my_i * HQ_SHARD, HQ_SHARD, axis=1)
    K = jnp.transpose(K, (1, 0, 2))
    V = jnp.transpose(V, (1, 0, 2))

    def body(x_ref, wq_ref, k_ref, v_ref, wo_ref, out_ref,
             comm_ref, send_sems, recv_sems, ctx_ref, xp_ref):
        my = lax.axis_index("i")
        left = lax.rem(my + N_DEV - 1, N_DEV)
        right = lax.rem(my + 1, N_DEV)

        barrier_sem = pltpu.get_barrier_semaphore()
        for nbr in (left, right):
            pl.semaphore_signal(barrier_sem, inc=1, device_id=(nbr,),
                                device_id_type=pl.DeviceIdType.MESH)
        pl.semaphore_wait(barrier_sem, 2)

        def cw_chunk(j):
            return lax.rem(my - j + N_DEV, N_DEV)

        def ccw_chunk(j):
            return N_DEV + lax.rem(my + j, N_DEV)

        P2 = 2 * CHUNK
        iota0 = lax.broadcasted_iota(jnp.int32, (P2, SKV), 0)
        cb2 = lax.broadcasted_iota(jnp.int32, (P2, SKV), 1) // BLK

        def compute_pair(j):
            ca = cw_chunk(j)
            cc = ccw_chunk(j)
            r0a = ca * CHUNK
            r0b = cc * CHUNK
            xp_ref[0:CHUNK, :] = x_ref[pl.ds(r0a, CHUNK), :]
            xp_ref[CHUNK:P2, :] = x_ref[pl.ds(r0b, CHUNK), :]
            qp = jnp.dot(xp_ref[...], wq_ref[...],
                         preferred_element_type=jnp.float32)
            roff = jnp.where(iota0 < CHUNK, r0a, r0b - CHUNK)
            mask = cb2 <= ((iota0 + roff) // BLK)
            for h in range(HQ_SHARD):
                qh = qp[:, h * DH:(h + 1) * DH]
                s = lax.dot_general(qh, k_ref[h],
                                    (((1,), (1,)), ((), ())),
                                    preferred_element_type=jnp.float32) * SCALE
                s = jnp.where(mask, s, -1e9)
                m = jnp.max(s, axis=-1, keepdims=True)
                w = jnp.exp(s - m)
                w = w / jnp.sum(w, axis=-1, keepdims=True)
                ctx_ref[:, h * DH:(h + 1) * DH] = jnp.dot(
                    w, v_ref[h], preferred_element_type=jnp.float32)
            op = jnp.dot(ctx_ref[...], wo_ref[...],
                         preferred_element_type=jnp.float32)
            out_ref[ca] = op[0:CHUNK, :]
            out_ref[cc] = op[CHUNK:P2, :]

        def make(src, slot, dst_dev):
            return pltpu.make_async_remote_copy(
                src_ref=src,
                dst_ref=comm_ref.at[slot],
                send_sem=send_sems.at[slot],
                recv_sem=recv_sems.at[slot],
                device_id=(dst_dev,),
                device_id_type=pl.DeviceIdType.MESH,
            )

        compute_pair(0)

        for hop in range(N_DEV - 1):
            cw = make(out_ref.at[cw_chunk(hop)], hop, right)
            ccw = make(out_ref.at[ccw_chunk(hop)], N_HOPS + hop, left)
            cw.start()
            ccw.start()
            compute_pair(hop + 1)
            cw.wait_recv()
            ccw.wait_recv()
            cw_recv = cw_chunk(hop + 1)
            ccw_recv = ccw_chunk(hop + 1)
            out_ref[cw_recv] = out_ref[cw_recv] + comm_ref[hop]
            out_ref[ccw_recv] = out_ref[ccw_recv] + comm_ref[N_HOPS + hop]

        for g in range(N_DEV - 1):
            hop = N_DEV - 1 + g
            if g == 0:
                cw_src = out_ref.at[cw_chunk(N_DEV - 1)]
                ccw_src = out_ref.at[ccw_chunk(N_DEV - 1)]
            else:
                cw_src = comm_ref.at[hop - 1]
                ccw_src = comm_ref.at[N_HOPS + hop - 1]
            cw = make(cw_src, hop, right)
            ccw = make(ccw_src, N_HOPS + hop, left)
            cw.start()
            ccw.start()
            cw.wait_recv()
            ccw.wait_recv()
            out_ref[lax.rem(my - g + N_DEV, N_DEV)] = comm_ref[hop]
            out_ref[N_DEV + lax.rem(my + g, N_DEV)] = comm_ref[N_HOPS + hop]

        for slot in range(2 * N_HOPS):
            drain = make(comm_ref.at[slot], slot, right)
            drain.wait_send()

    out = pl.pallas_call(
        body,
        out_shape=jax.ShapeDtypeStruct((N_CHUNKS, CHUNK, SQ), jnp.float32),
        in_specs=[pl.BlockSpec(memory_space=pltpu.VMEM)] * 5,
        out_specs=pl.BlockSpec(memory_space=pltpu.VMEM),
        scratch_shapes=[
            pltpu.VMEM((2 * N_HOPS, CHUNK, SQ), jnp.float32),
            pltpu.SemaphoreType.DMA((2 * N_HOPS,)),
            pltpu.SemaphoreType.DMA((2 * N_HOPS,)),
            pltpu.VMEM((2 * CHUNK, HQ_SHARD * DH), jnp.float32),
            pltpu.VMEM((2 * CHUNK, SQ), jnp.float32),
        ],
        compiler_params=pltpu.CompilerParams(collective_id=0),
    )(x2, Wq, K, V, Wo)
    return out.reshape(1, SQ, SQ)


# device time: 61074 ns/iter; 1.8575x vs baseline; 1.1168x over previous
import os

import jax
import jax.numpy as jnp
from jax import lax
from jax.experimental import pallas as pl
from jax.experimental.pallas import tpu as pltpu

N_DEV = 4
SQ = 1024
SKV = 1024
HQ_SHARD = 8
DH = 128
BLK = 64
SCALE = 0.08838834764831843
N_CHUNKS = 2 * N_DEV
CHUNK = SQ // N_CHUNKS
N_HOPS = 2 * (N_DEV - 1)
P2 = 2 * CHUNK

_SKIP_COMM = bool(os.environ.get("SKIP_COMM"))


def kernel(x, Wq, K_ext, V_ext, Wo):
    my_i = lax.axis_index("i")
    xb = x[0].astype(jnp.bfloat16)
    wqb = Wq.astype(jnp.bfloat16)
    wob = Wo.astype(jnp.bfloat16)
    K = lax.dynamic_slice_in_dim(K_ext[0], my_i * HQ_SHARD, HQ_SHARD, axis=1)
    V = lax.dynamic_slice_in_dim(V_ext[0], my_i * HQ_SHARD, HQ_SHARD, axis=1)
    K = jnp.transpose(K, (1, 0, 2)).astype(jnp.bfloat16)
    V = jnp.transpose(V, (1, 0, 2)).astype(jnp.bfloat16)

    def body(x_ref, wq_ref, k_ref, v_ref, wo_ref, out_ref,
             acc_ref, comm_ref, send_sems, recv_sems, ctx_ref, xp_ref):
        my = lax.axis_index("i")
        left = lax.rem(my + N_DEV - 1, N_DEV)
        right = lax.rem(my + 1, N_DEV)

        barrier_sem = pltpu.get_barrier_semaphore()
        for nbr in (left, right):
            pl.semaphore_signal(barrier_sem, inc=1, device_id=(nbr,),
                                device_id_type=pl.DeviceIdType.MESH)
        pl.semaphore_wait(barrier_sem, 2)

        def cw_chunk(j):
            return lax.rem(my - j + N_DEV, N_DEV)

        def ccw_chunk(j):
            return N_DEV + lax.rem(my + j, N_DEV)

        iota_col = lax.broadcasted_iota(jnp.int32, (P2, 1), 0)
        cb2 = lax.broadcasted_iota(jnp.int32, (P2, SKV), 1) // BLK

        def compute_pair(j):
            ca = cw_chunk(j)
            cc = ccw_chunk(j)
            r0a = ca * CHUNK
            r0b = cc * CHUNK
            xp_ref[0:CHUNK, :] = x_ref[pl.ds(r0a, CHUNK), :]
            xp_ref[CHUNK:P2, :] = x_ref[pl.ds(r0b, CHUNK), :]
            qp = jnp.dot(xp_ref[...], wq_ref[...],
                         preferred_element_type=jnp.float32)
            qb = qp.astype(jnp.bfloat16)
            roff = jnp.where(iota_col < CHUNK, r0a, r0b - CHUNK)
            rb = (iota_col + roff) // BLK
            mask = cb2 <= rb
            for h in range(HQ_SHARD):
                qh = qb[:, h * DH:(h + 1) * DH]
                s = lax.dot_general(qh, k_ref[h],
                                    (((1,), (1,)), ((), ())),
                                    preferred_element_type=jnp.float32) * SCALE
                s = jnp.where(mask, s, -1e9)
                m = jnp.max(s, axis=-1, keepdims=True)
                w = jnp.exp(s - m)
                w = w / jnp.sum(w, axis=-1, keepdims=True)
                ctx_ref[:, h * DH:(h + 1) * DH] = (
                    jnp.dot(w.astype(jnp.bfloat16), v_ref[h],
                            preferred_element_type=jnp.float32)
                ).astype(jnp.bfloat16)
            op = jnp.dot(ctx_ref[...], wo_ref[...],
                         preferred_element_type=jnp.float32)
            acc_ref[ca] = op[0:CHUNK, :].astype(jnp.bfloat16)
            acc_ref[cc] = op[CHUNK:P2, :].astype(jnp.bfloat16)

        if _SKIP_COMM:
            for j in range(N_DEV):
                compute_pair(j)
            for c in range(N_CHUNKS):
                out_ref[c] = acc_ref[c].astype(jnp.float32)
            return

        def make(src, slot, dst_dev):
            return pltpu.make_async_remote_copy(
                src_ref=src,
                dst_ref=comm_ref.at[slot],
                send_sem=send_sems.at[slot],
                recv_sem=recv_sems.at[slot],
                device_id=(dst_dev,),
                device_id_type=pl.DeviceIdType.MESH,
            )

        compute_pair(0)

        for hop in range(N_DEV - 1):
            cw = make(acc_ref.at[cw_chunk(hop)], hop, right)
            ccw = make(acc_ref.at[ccw_chunk(hop)], N_HOPS + hop, left)
            cw.start()
            ccw.start()
            compute_pair(hop + 1)
            cw.wait_recv()
            ccw.wait_recv()
            cw_recv = cw_chunk(hop + 1)
            ccw_recv = ccw_chunk(hop + 1)
            acc_ref[cw_recv] = acc_ref[cw_recv] + comm_ref[hop]
            acc_ref[ccw_recv] = acc_ref[ccw_recv] + comm_ref[N_HOPS + hop]

        for g in range(N_DEV - 1):
            hop = N_DEV - 1 + g
            if g == 0:
                cw_src = acc_ref.at[cw_chunk(N_DEV - 1)]
                ccw_src = acc_ref.at[ccw_chunk(N_DEV - 1)]
            else:
                cw_src = comm_ref.at[hop - 1]
                ccw_src = comm_ref.at[N_HOPS + hop - 1]
            cw = make(cw_src, hop, right)
            ccw = make(ccw_src, N_HOPS + hop, left)
            cw.start()
            ccw.start()
            if g == 0:
                cwc = cw_chunk(N_DEV - 1)
                ccwc = ccw_chunk(N_DEV - 1)
                out_ref[cwc] = acc_ref[cwc].astype(jnp.float32)
                out_ref[ccwc] = acc_ref[ccwc].astype(jnp.float32)
            cw.wait_recv()
            ccw.wait_recv()
            out_ref[lax.rem(my - g + N_DEV, N_DEV)] = (
                comm_ref[hop].astype(jnp.float32))
            out_ref[N_DEV + lax.rem(my + g, N_DEV)] = (
                comm_ref[N_HOPS + hop].astype(jnp.float32))

        for slot in range(2 * N_HOPS):
            drain = make(comm_ref.at[slot], slot, right)
            drain.wait_send()

    out = pl.pallas_call(
        body,
        out_shape=jax.ShapeDtypeStruct((N_CHUNKS, CHUNK, SQ), jnp.float32),
        in_specs=[pl.BlockSpec(memory_space=pltpu.VMEM)] * 5,
        out_specs=pl.BlockSpec(memory_space=pltpu.VMEM),
        scratch_shapes=[
            pltpu.VMEM((N_CHUNKS, CHUNK, SQ), jnp.bfloat16),
            pltpu.VMEM((2 * N_HOPS, CHUNK, SQ), jnp.bfloat16),
            pltpu.SemaphoreType.DMA((2 * N_HOPS,)),
            pltpu.SemaphoreType.DMA((2 * N_HOPS,)),
            pltpu.VMEM((P2, HQ_SHARD * DH), jnp.bfloat16),
            pltpu.VMEM((P2, SQ), jnp.bfloat16),
        ],
        compiler_params=pltpu.CompilerParams(collective_id=0),
    )(xb, wqb, K, V, wob)
    return out.reshape(1, SQ, SQ)


# device time: 49494 ns/iter; 2.2922x vs baseline; 1.2340x over previous
import os

import jax
import jax.numpy as jnp
from jax import lax
from jax.experimental import pallas as pl
from jax.experimental.pallas import tpu as pltpu

N_DEV = 4
SQ = 1024
SKV = 1024
HQ_SHARD = 8
DH = 128
BLK = 64
SCALE = 0.08838834764831843
N_CHUNKS = 2 * N_DEV
CHUNK = SQ // N_CHUNKS
N_HOPS = 2 * (N_DEV - 1)
P2 = 2 * CHUNK

_SKIP_COMM = bool(os.environ.get("SKIP_COMM"))


def kernel(x, Wq, K_ext, V_ext, Wo):
    my_i = lax.axis_index("i")
    x2 = x[0]
    wqb = (Wq * SCALE).astype(jnp.bfloat16)
    wob = Wo.astype(jnp.bfloat16)
    K = lax.dynamic_slice_in_dim(K_ext[0], my_i * HQ_SHARD, HQ_SHARD, axis=1)
    V = lax.dynamic_slice_in_dim(V_ext[0], my_i * HQ_SHARD, HQ_SHARD, axis=1)
    Kb = K.astype(jnp.bfloat16).reshape(SKV, HQ_SHARD * DH)
    Vb = V.astype(jnp.bfloat16).reshape(SKV, HQ_SHARD * DH)

    def body(x_ref, wq_ref, k_ref, v_ref, wo_ref, out_ref,
             acc_ref, comm_ref, send_sems, recv_sems, ctx_ref, xp_ref):
        my = lax.axis_index("i")
        left = lax.rem(my + N_DEV - 1, N_DEV)
        right = lax.rem(my + 1, N_DEV)

        barrier_sem = pltpu.get_barrier_semaphore()
        for nbr in (left, right):
            pl.semaphore_signal(barrier_sem, inc=1, device_id=(nbr,),
                                device_id_type=pl.DeviceIdType.MESH)
        pl.semaphore_wait(barrier_sem, 2)

        def cw_chunk(j):
            return lax.rem(my - j + N_DEV, N_DEV)

        def ccw_chunk(j):
            return N_DEV + lax.rem(my + j, N_DEV)

        iota_col = lax.broadcasted_iota(jnp.int32, (P2, 1), 0)
        cb2 = lax.broadcasted_iota(jnp.int32, (P2, SKV), 1) // BLK

        def compute_pair(j):
            ca = cw_chunk(j)
            cc = ccw_chunk(j)
            r0a = ca * CHUNK
            r0b = cc * CHUNK
            xp_ref[0:CHUNK, :] = x_ref[pl.ds(r0a, CHUNK), :].astype(
                jnp.bfloat16)
            xp_ref[CHUNK:P2, :] = x_ref[pl.ds(r0b, CHUNK), :].astype(
                jnp.bfloat16)
            qp = jnp.dot(xp_ref[...], wq_ref[...],
                         preferred_element_type=jnp.float32)
            qb = qp.astype(jnp.bfloat16)
            roff = jnp.where(iota_col < CHUNK, r0a, r0b - CHUNK)
            rb = (iota_col + roff) // BLK
            bias = jnp.where(cb2 <= rb, 0.0, -1e9)
            for h in range(HQ_SHARD):
                qh = qb[:, h * DH:(h + 1) * DH]
                kh = k_ref[:, h * DH:(h + 1) * DH]
                vh = v_ref[:, h * DH:(h + 1) * DH]
                s = lax.dot_general(qh, kh,
                                    (((1,), (1,)), ((), ())),
                                    preferred_element_type=jnp.float32)
                w = jnp.exp(s + bias)
                inv = 1.0 / jnp.sum(w, axis=-1, keepdims=True)
                ctx = jnp.dot(w.astype(jnp.bfloat16), vh,
                              preferred_element_type=jnp.float32)
                ctx_ref[:, h * DH:(h + 1) * DH] = (ctx * inv).astype(
                    jnp.bfloat16)
            op = jnp.dot(ctx_ref[...], wo_ref[...],
                         preferred_element_type=jnp.float32)
            acc_ref[ca] = op[0:CHUNK, :].astype(jnp.bfloat16)
            acc_ref[cc] = op[CHUNK:P2, :].astype(jnp.bfloat16)

        if _SKIP_COMM:
            for j in range(N_DEV):
                compute_pair(j)
            for c in range(N_CHUNKS):
                out_ref[c] = acc_ref[c].astype(jnp.float32)
            return

        def make(src, slot, dst_dev):
            return pltpu.make_async_remote_copy(
                src_ref=src,
                dst_ref=comm_ref.at[slot],
                send_sem=send_sems.at[slot],
                recv_sem=recv_sems.at[slot],
                device_id=(dst_dev,),
                device_id_type=pl.DeviceIdType.MESH,
            )

        compute_pair(0)

        for hop in range(N_DEV - 1):
            cw = make(acc_ref.at[cw_chunk(hop)], hop, right)
            ccw = make(acc_ref.at[ccw_chunk(hop)], N_HOPS + hop, left)
            cw.start()
            ccw.start()
            compute_pair(hop + 1)
            cw.wait_recv()
            ccw.wait_recv()
            cw_recv = cw_chunk(hop + 1)
            ccw_recv = ccw_chunk(hop + 1)
            acc_ref[cw_recv] = acc_ref[cw_recv] + comm_ref[hop]
            acc_ref[ccw_recv] = acc_ref[ccw_recv] + comm_ref[N_HOPS + hop]

        for g in range(N_DEV - 1):
            hop = N_DEV - 1 + g
            if g == 0:
                cw_src = acc_ref.at[cw_chunk(N_DEV - 1)]
                ccw_src = acc_ref.at[ccw_chunk(N_DEV - 1)]
            else:
                cw_src = comm_ref.at[hop - 1]
                ccw_src = comm_ref.at[N_HOPS + hop - 1]
            cw = make(cw_src, hop, right)
            ccw = make(ccw_src, N_HOPS + hop, left)
            cw.start()
            ccw.start()
            if g == 0:
                cwc = cw_chunk(N_DEV - 1)
                ccwc = ccw_chunk(N_DEV - 1)
                out_ref[cwc] = acc_ref[cwc].astype(jnp.float32)
                out_ref[ccwc] = acc_ref[ccwc].astype(jnp.float32)
            cw.wait_recv()
            ccw.wait_recv()
            out_ref[lax.rem(my - g + N_DEV, N_DEV)] = (
                comm_ref[hop].astype(jnp.float32))
            out_ref[N_DEV + lax.rem(my + g, N_DEV)] = (
                comm_ref[N_HOPS + hop].astype(jnp.float32))

        for slot in range(2 * N_HOPS):
            drain = make(comm_ref.at[slot], slot, right)
            drain.wait_send()

    out = pl.pallas_call(
        body,
        out_shape=jax.ShapeDtypeStruct((N_CHUNKS, CHUNK, SQ), jnp.float32),
        in_specs=[pl.BlockSpec(memory_space=pltpu.VMEM)] * 5,
        out_specs=pl.BlockSpec(memory_space=pltpu.VMEM),
        scratch_shapes=[
            pltpu.VMEM((N_CHUNKS, CHUNK, SQ), jnp.bfloat16),
            pltpu.VMEM((2 * N_HOPS, CHUNK, SQ), jnp.bfloat16),
            pltpu.SemaphoreType.DMA((2 * N_HOPS,)),
            pltpu.SemaphoreType.DMA((2 * N_HOPS,)),
            pltpu.VMEM((P2, HQ_SHARD * DH), jnp.bfloat16),
            pltpu.VMEM((P2, SQ), jnp.bfloat16),
        ],
        compiler_params=pltpu.CompilerParams(collective_id=0),
    )(x2, wqb, Kb, Vb, wob)
    return out.reshape(1, SQ, SQ)


# device time: 47503 ns/iter; 2.3882x vs baseline; 1.0419x over previous
import os

import jax
import jax.numpy as jnp
from jax import lax
from jax.experimental import pallas as pl
from jax.experimental.pallas import tpu as pltpu

N_DEV = 4
SQ = 1024
SKV = 1024
HQ_SHARD = 8
DH = 128
BLK = 64
SCALE = 0.08838834764831843
N_CHUNKS = 2 * N_DEV
CHUNK = SQ // N_CHUNKS
N_HOPS = 2 * (N_DEV - 1)
P2 = 2 * CHUNK

_SKIP_COMM = bool(os.environ.get("SKIP_COMM"))


def kernel(x, Wq, K_ext, V_ext, Wo):
    my_i = lax.axis_index("i")
    x2 = x[0]
    K = lax.dynamic_slice_in_dim(K_ext[0], my_i * HQ_SHARD, HQ_SHARD, axis=1)
    V = lax.dynamic_slice_in_dim(V_ext[0], my_i * HQ_SHARD, HQ_SHARD, axis=1)
    Kb = K.astype(jnp.bfloat16).reshape(SKV, HQ_SHARD * DH)
    Vb = V.astype(jnp.bfloat16).reshape(SKV, HQ_SHARD * DH)

    def body(x_ref, wq_ref, k_ref, v_ref, wo_ref, out_ref,
             acc_ref, comm_ref, send_sems, recv_sems, ctx_ref, xp_ref,
             wqb_ref, wob_ref):
        my = lax.axis_index("i")
        left = lax.rem(my + N_DEV - 1, N_DEV)
        right = lax.rem(my + 1, N_DEV)

        barrier_sem = pltpu.get_barrier_semaphore()
        for nbr in (left, right):
            pl.semaphore_signal(barrier_sem, inc=1, device_id=(nbr,),
                                device_id_type=pl.DeviceIdType.MESH)
        pl.semaphore_wait(barrier_sem, 2)

        wqb_ref[...] = (wq_ref[...] * SCALE).astype(jnp.bfloat16)
        wob_ref[...] = wo_ref[...].astype(jnp.bfloat16)

        def cw_chunk(j):
            return lax.rem(my - j + N_DEV, N_DEV)

        def ccw_chunk(j):
            return N_DEV + lax.rem(my + j, N_DEV)

        iota_col = lax.broadcasted_iota(jnp.int32, (P2, 1), 0)
        cb2 = lax.broadcasted_iota(jnp.int32, (P2, SKV), 1) // BLK

        def compute_pair(j):
            ca = cw_chunk(j)
            cc = ccw_chunk(j)
            r0a = ca * CHUNK
            r0b = cc * CHUNK
            xp_ref[0:CHUNK, :] = x_ref[pl.ds(r0a, CHUNK), :].astype(
                jnp.bfloat16)
            xp_ref[CHUNK:P2, :] = x_ref[pl.ds(r0b, CHUNK), :].astype(
                jnp.bfloat16)
            qp = jnp.dot(xp_ref[...], wqb_ref[...],
                         preferred_element_type=jnp.float32)
            qb = qp.astype(jnp.bfloat16)
            roff = jnp.where(iota_col < CHUNK, r0a, r0b - CHUNK)
            rb = (iota_col + roff) // BLK
            bias = jnp.where(cb2 <= rb, 0.0, -1e9)
            for h in range(HQ_SHARD):
                qh = qb[:, h * DH:(h + 1) * DH]
                kh = k_ref[:, h * DH:(h + 1) * DH]
                vh = v_ref[:, h * DH:(h + 1) * DH]
                s = lax.dot_general(qh, kh,
                                    (((1,), (1,)), ((), ())),
                                    preferred_element_type=jnp.float32)
                w = jnp.exp(s + bias)
                inv = 1.0 / jnp.sum(w, axis=-1, keepdims=True)
                ctx = jnp.dot(w.astype(jnp.bfloat16), vh,
                              preferred_element_type=jnp.float32)
                ctx_ref[:, h * DH:(h + 1) * DH] = (ctx * inv).astype(
                    jnp.bfloat16)
            op = jnp.dot(ctx_ref[...], wob_ref[...],
                         preferred_element_type=jnp.float32)
            acc_ref[ca] = op[0:CHUNK, :].astype(jnp.bfloat16)
            acc_ref[cc] = op[CHUNK:P2, :].astype(jnp.bfloat16)

        if _SKIP_COMM:
            for j in range(N_DEV):
                compute_pair(j)
            for c in range(N_CHUNKS):
                out_ref[c] = acc_ref[c].astype(jnp.float32)
            return

        def make(src, slot, dst_dev):
            return pltpu.make_async_remote_copy(
                src_ref=src,
                dst_ref=comm_ref.at[slot],
                send_sem=send_sems.at[slot],
                recv_sem=recv_sems.at[slot],
                device_id=(dst_dev,),
                device_id_type=pl.DeviceIdType.MESH,
            )

        compute_pair(0)

        for hop in range(N_DEV - 1):
            cw = make(acc_ref.at[cw_chunk(hop)], hop, right)
            ccw = make(acc_ref.at[ccw_chunk(hop)], N_HOPS + hop, left)
            cw.start()
            ccw.start()
            compute_pair(hop + 1)
            cw.wait_recv()
            ccw.wait_recv()
            cw_recv = cw_chunk(hop + 1)
            ccw_recv = ccw_chunk(hop + 1)
            acc_ref[cw_recv] = acc_ref[cw_recv] + comm_ref[hop]
            acc_ref[ccw_recv] = acc_ref[ccw_recv] + comm_ref[N_HOPS + hop]

        for g in range(N_DEV - 1):
            hop = N_DEV - 1 + g
            if g == 0:
                cw_src = acc_ref.at[cw_chunk(N_DEV - 1)]
                ccw_src = acc_ref.at[ccw_chunk(N_DEV - 1)]
            else:
                cw_src = comm_ref.at[hop - 1]
                ccw_src = comm_ref.at[N_HOPS + hop - 1]
            cw = make(cw_src, hop, right)
            ccw = make(ccw_src, N_HOPS + hop, left)
            cw.start()
            ccw.start()
            if g == 0:
                cwc = cw_chunk(N_DEV - 1)
                ccwc = ccw_chunk(N_DEV - 1)
                out_ref[cwc] = acc_ref[cwc].astype(jnp.float32)
                out_ref[ccwc] = acc_ref[ccwc].astype(jnp.float32)
            cw.wait_recv()
            ccw.wait_recv()
            out_ref[lax.rem(my - g + N_DEV, N_DEV)] = (
                comm_ref[hop].astype(jnp.float32))
            out_ref[N_DEV + lax.rem(my + g, N_DEV)] = (
                comm_ref[N_HOPS + hop].astype(jnp.float32))

        for slot in range(2 * N_HOPS):
            drain = make(comm_ref.at[slot], slot, right)
            drain.wait_send()

    out = pl.pallas_call(
        body,
        out_shape=jax.ShapeDtypeStruct((N_CHUNKS, CHUNK, SQ), jnp.float32),
        in_specs=[pl.BlockSpec(memory_space=pltpu.VMEM)] * 5,
        out_specs=pl.BlockSpec(memory_space=pltpu.VMEM),
        scratch_shapes=[
            pltpu.VMEM((N_CHUNKS, CHUNK, SQ), jnp.bfloat16),
            pltpu.VMEM((2 * N_HOPS, CHUNK, SQ), jnp.bfloat16),
            pltpu.SemaphoreType.DMA((2 * N_HOPS,)),
            pltpu.SemaphoreType.DMA((2 * N_HOPS,)),
            pltpu.VMEM((P2, HQ_SHARD * DH), jnp.bfloat16),
            pltpu.VMEM((P2, SQ), jnp.bfloat16),
            pltpu.VMEM((SQ, HQ_SHARD * DH), jnp.bfloat16),
            pltpu.VMEM((HQ_SHARD * DH, SQ), jnp.bfloat16),
        ],
        compiler_params=pltpu.CompilerParams(collective_id=0),
    )(x2, Wq, Kb, Vb, Wo)
    return out.reshape(1, SQ, SQ)
